# Initial kernel scaffold; baseline (speedup 1.0000x reference)
import functools

import jax
import jax.numpy as jnp
from jax import lax
from jax.experimental import pallas as pl
from jax.experimental.pallas import tpu as pltpu
from jax.experimental.pallas import tpu_sc as plsc

N = 10000
E = 160000
D_IN = 256
D_HID = 128
D_OUT = 256

NP = 10240
DUMMY = N
NW = 32
TK = 40
B = 128
EP = NW * TK * B
RPT = NP // 16


def _make_seg_sum(with_deg: bool):
  mesh = plsc.VectorSubcoreMesh(core_axis_name="c", subcore_axis_name="s")
  out_type = [jax.ShapeDtypeStruct((2, NP, D_HID), jnp.float32)]
  if with_deg:
    out_type.append(jax.ShapeDtypeStruct((2, NP), jnp.float32))
  scratch = [
      pltpu.VMEM((TK, B), jnp.int32),
      pltpu.VMEM((TK, B), jnp.int32),
      pltpu.VMEM((B, D_HID), jnp.float32),
      pltpu.VMEM((B,), jnp.float32),
      pltpu.VMEM((B,), jnp.float32),
      pltpu.VMEM_SHARED((NP, D_HID), jnp.float32),
      pltpu.VMEM_SHARED((NP,), jnp.float32),
      pltpu.SemaphoreType.DMA,
  ]

  def body(y_hbm, src_hbm, dst_hbm, *rest):
    if with_deg:
      (out_rows, out_deg, src_v, dst_v, rows_v, ones_v, zeros_v,
       acc_sh, deg_sh, sem) = rest
    else:
      (out_rows, src_v, dst_v, rows_v, ones_v, zeros_v,
       acc_sh, deg_sh, sem) = rest
      out_deg = None
    cid = lax.axis_index("c")
    sid = lax.axis_index("s")
    w = sid * 2 + cid

    one16 = jnp.ones((16,), jnp.float32)
    zero16 = jnp.zeros((16,), jnp.float32)
    for jj in range(B // 16):
      ones_v[pl.ds(jj * 16, 16)] = one16
      zeros_v[pl.ds(jj * 16, 16)] = zero16

    def zero_rows(i, carry):
      for jj in range(D_HID // 16):
        rows_v[i, pl.ds(jj * 16, 16)] = zero16
      return carry
    lax.fori_loop(0, B, zero_rows, 0)

    base = sid * RPT
    for bb in range(RPT // B):
      pltpu.sync_copy(rows_v, acc_sh.at[pl.ds(base + bb * B, B)])
      pltpu.sync_copy(zeros_v, deg_sh.at[pl.ds(base + bb * B, B)])

    pltpu.sync_copy(src_hbm.at[w], src_v)
    pltpu.sync_copy(dst_hbm.at[w], dst_v)
    plsc.subcore_barrier()

    def step(j, carry):
      pltpu.async_copy(y_hbm.at[src_v.at[j]], rows_v, sem).wait()
      pltpu.sync_copy(rows_v, acc_sh.at[dst_v.at[j]], add=True)
      if with_deg:
        pltpu.sync_copy(ones_v, deg_sh.at[dst_v.at[j]], add=True)
      return carry
    lax.fori_loop(0, TK, step, 0)

    plsc.subcore_barrier()
    pltpu.sync_copy(acc_sh.at[pl.ds(base, RPT)],
                    out_rows.at[cid, pl.ds(base, RPT)])
    if with_deg:
      pltpu.sync_copy(deg_sh.at[pl.ds(base, RPT)],
                      out_deg.at[cid, pl.ds(base, RPT)])

  return pl.kernel(body, mesh=mesh, out_type=out_type, scratch_types=scratch)


_seg_sum_deg = _make_seg_sum(with_deg=True)
_seg_sum = _make_seg_sum(with_deg=False)


def _proj_body(x_ref, wl_ref, wr_ref, b1_ref, y_ref, z_ref):
  x = x_ref[...]
  y_ref[...] = jnp.dot(x, wl_ref[...], preferred_element_type=jnp.float32)
  z_ref[...] = (jnp.dot(x, wr_ref[...], preferred_element_type=jnp.float32)
                + b1_ref[...])


def _layer1_proj(x, W1_l, W1_r, b1):
  blk = 400
  grid = N // blk
  return pl.pallas_call(
      _proj_body,
      grid=(grid,),
      in_specs=[
          pl.BlockSpec((blk, D_IN), lambda i: (i, 0)),
          pl.BlockSpec((D_IN, D_HID), lambda i: (0, 0)),
          pl.BlockSpec((D_IN, D_HID), lambda i: (0, 0)),
          pl.BlockSpec((1, D_HID), lambda i: (0, 0)),
      ],
      out_specs=[
          pl.BlockSpec((blk, D_HID), lambda i: (i, 0)),
          pl.BlockSpec((blk, D_HID), lambda i: (i, 0)),
      ],
      out_shape=[
          jax.ShapeDtypeStruct((NP, D_HID), jnp.float32),
          jax.ShapeDtypeStruct((NP, D_HID), jnp.float32),
      ],
  )(x, W1_l, W1_r, b1.reshape(1, D_HID))


def _combine1_body(p_ref, dg_ref, z_ref, h_ref):
  s = p_ref[0] + p_ref[1]
  d = dg_ref[0] + dg_ref[1]
  inv = 1.0 / jnp.maximum(d, 1.0)
  h_ref[...] = jnp.maximum(s * inv + z_ref[...], 0.0)


def _combine1(p, deg_col, z):
  blk = 512
  grid = NP // blk
  return pl.pallas_call(
      _combine1_body,
      grid=(grid,),
      in_specs=[
          pl.BlockSpec((2, blk, D_HID), lambda i: (0, i, 0)),
          pl.BlockSpec((2, blk, 1), lambda i: (0, i, 0)),
          pl.BlockSpec((blk, D_HID), lambda i: (i, 0)),
      ],
      out_specs=pl.BlockSpec((blk, D_HID), lambda i: (i, 0)),
      out_shape=jax.ShapeDtypeStruct((NP, D_HID), jnp.float32),
  )(p, deg_col, z)


def _layer2_body(q_ref, dg_ref, h_ref, wl_ref, wr_ref, b2_ref, o_ref):
  d = dg_ref[0] + dg_ref[1]
  inv = 1.0 / jnp.maximum(d, 1.0)
  agg = (q_ref[0] + q_ref[1]) * inv
  o_ref[...] = (jnp.dot(agg, wl_ref[...], preferred_element_type=jnp.float32)
                + jnp.dot(h_ref[...], wr_ref[...],
                          preferred_element_type=jnp.float32)
                + b2_ref[...])


def _layer2(q, deg_col, h, W2_l, W2_r, b2):
  blk = 512
  grid = NP // blk
  return pl.pallas_call(
      _layer2_body,
      grid=(grid,),
      in_specs=[
          pl.BlockSpec((2, blk, D_HID), lambda i: (0, i, 0)),
          pl.BlockSpec((2, blk, 1), lambda i: (0, i, 0)),
          pl.BlockSpec((blk, D_HID), lambda i: (i, 0)),
          pl.BlockSpec((D_HID, D_OUT), lambda i: (0, 0)),
          pl.BlockSpec((D_HID, D_OUT), lambda i: (0, 0)),
          pl.BlockSpec((1, D_OUT), lambda i: (0, 0)),
      ],
      out_specs=pl.BlockSpec((blk, D_OUT), lambda i: (i, 0)),
      out_shape=jax.ShapeDtypeStruct((NP, D_OUT), jnp.float32),
  )(q, deg_col, h, W2_l, W2_r, b2.reshape(1, D_OUT))


def kernel(x, edge_index, W1_l, b1, W1_r, W2_l, b2, W2_r):
  src = edge_index[0].astype(jnp.int32)
  dst = edge_index[1].astype(jnp.int32)
  pad = EP - E
  srcp = jnp.concatenate([src, jnp.zeros((pad,), jnp.int32)]).reshape(NW, TK, B)
  dstp = jnp.concatenate([dst, jnp.full((pad,), DUMMY, jnp.int32)]
                         ).reshape(NW, TK, B)

  y1, z1 = _layer1_proj(x, W1_l, W1_r, b1)
  p1, deg = _seg_sum_deg(y1, srcp, dstp)
  deg_col = deg.reshape(2, NP, 1)
  h = _combine1(p1, deg_col, z1)
  p2 = _seg_sum(h, srcp, dstp)
  out = _layer2(p2, deg_col, h, W2_l, W2_r, b2)
  return out[:N]


# speedup vs baseline: 3.4454x; 3.4454x over previous
import functools

import jax
import jax.numpy as jnp
from jax import lax
from jax.experimental import pallas as pl
from jax.experimental.pallas import tpu as pltpu
from jax.experimental.pallas import tpu_sc as plsc

N = 10000
E = 160000
D_IN = 256
D_HID = 128
D_OUT = 256

NP = 10240
DUMMY = N
NW = 32
TK = 40
B = 128
EP = NW * TK * B
RPT = NP // 16


def _make_seg_sum(with_deg: bool):
  mesh = plsc.VectorSubcoreMesh(core_axis_name="c", subcore_axis_name="s")
  out_type = [jax.ShapeDtypeStruct((2, NP, D_HID), jnp.float32)]
  if with_deg:
    out_type.append(jax.ShapeDtypeStruct((2, NP), jnp.float32))
  scratch = [
      pltpu.VMEM((TK, B), jnp.int32),
      pltpu.VMEM((TK, B), jnp.int32),
      pltpu.VMEM((B, D_HID), jnp.float32),
      pltpu.VMEM((B,), jnp.float32),
      pltpu.VMEM((B,), jnp.float32),
      pltpu.VMEM_SHARED((NP, D_HID), jnp.float32),
      pltpu.VMEM_SHARED((NP,), jnp.float32),
      pltpu.SemaphoreType.DMA,
  ]

  def body(y_hbm, src_hbm, dst_hbm, *rest):
    if with_deg:
      (out_rows, out_deg, src_v, dst_v, rows_v, ones_v, zeros_v,
       acc_sh, deg_sh, sem) = rest
    else:
      (out_rows, src_v, dst_v, rows_v, ones_v, zeros_v,
       acc_sh, deg_sh, sem) = rest
      out_deg = None
    cid = lax.axis_index("c")
    sid = lax.axis_index("s")
    w = sid * 2 + cid

    one16 = jnp.ones((16,), jnp.float32)
    zero16 = jnp.zeros((16,), jnp.float32)
    for jj in range(B // 16):
      ones_v[pl.ds(jj * 16, 16)] = one16
      zeros_v[pl.ds(jj * 16, 16)] = zero16

    def zero_rows(i, carry):
      for jj in range(D_HID // 16):
        rows_v[i, pl.ds(jj * 16, 16)] = zero16
      return carry
    lax.fori_loop(0, B, zero_rows, 0)

    base = sid * RPT
    for bb in range(RPT // B):
      pltpu.sync_copy(rows_v, acc_sh.at[pl.ds(base + bb * B, B)])
      pltpu.sync_copy(zeros_v, deg_sh.at[pl.ds(base + bb * B, B)])

    pltpu.sync_copy(src_hbm.at[w], src_v)
    pltpu.sync_copy(dst_hbm.at[w], dst_v)
    plsc.subcore_barrier()

    def step(j, carry):
      pltpu.async_copy(y_hbm.at[src_v.at[j]], rows_v, sem).wait()
      pltpu.sync_copy(rows_v, acc_sh.at[dst_v.at[j]], add=True)
      if with_deg:
        pltpu.sync_copy(ones_v, deg_sh.at[dst_v.at[j]], add=True)
      return carry
    lax.fori_loop(0, TK, step, 0)

    plsc.subcore_barrier()
    pltpu.sync_copy(acc_sh.at[pl.ds(base, RPT)],
                    out_rows.at[cid, pl.ds(base, RPT)])
    if with_deg:
      pltpu.sync_copy(deg_sh.at[pl.ds(base, RPT)],
                      out_deg.at[cid, pl.ds(base, RPT)])

  return pl.kernel(body, mesh=mesh, out_type=out_type, scratch_types=scratch)


_seg_sum_deg = _make_seg_sum(with_deg=True)
_seg_sum = _make_seg_sum(with_deg=False)


def _proj_body(x_ref, wl_ref, wr_ref, b1_ref, y_ref, z_ref):
  x = x_ref[...]
  y_ref[...] = jnp.dot(x, wl_ref[...], preferred_element_type=jnp.float32)
  z_ref[...] = (jnp.dot(x, wr_ref[...], preferred_element_type=jnp.float32)
                + b1_ref[...])


def _layer1_proj(x, W1_l, W1_r, b1):
  blk = 400
  grid = N // blk
  return pl.pallas_call(
      _proj_body,
      grid=(grid,),
      in_specs=[
          pl.BlockSpec((blk, D_IN), lambda i: (i, 0)),
          pl.BlockSpec((D_IN, D_HID), lambda i: (0, 0)),
          pl.BlockSpec((D_IN, D_HID), lambda i: (0, 0)),
          pl.BlockSpec((1, D_HID), lambda i: (0, 0)),
      ],
      out_specs=[
          pl.BlockSpec((blk, D_HID), lambda i: (i, 0)),
          pl.BlockSpec((blk, D_HID), lambda i: (i, 0)),
      ],
      out_shape=[
          jax.ShapeDtypeStruct((NP, D_HID), jnp.float32),
          jax.ShapeDtypeStruct((NP, D_HID), jnp.float32),
      ],
  )(x, W1_l, W1_r, b1.reshape(1, D_HID))


def _combine1_body(p_ref, dg_ref, z_ref, h_ref):
  s = p_ref[0] + p_ref[1]
  d = dg_ref[0] + dg_ref[1]
  inv = 1.0 / jnp.maximum(d, 1.0)
  h_ref[...] = jnp.maximum(s * inv + z_ref[...], 0.0)


def _combine1(p, deg_col, z):
  blk = 512
  grid = NP // blk
  return pl.pallas_call(
      _combine1_body,
      grid=(grid,),
      in_specs=[
          pl.BlockSpec((2, blk, D_HID), lambda i: (0, i, 0)),
          pl.BlockSpec((2, blk, 1), lambda i: (0, i, 0)),
          pl.BlockSpec((blk, D_HID), lambda i: (i, 0)),
      ],
      out_specs=pl.BlockSpec((blk, D_HID), lambda i: (i, 0)),
      out_shape=jax.ShapeDtypeStruct((NP, D_HID), jnp.float32),
  )(p, deg_col, z)


def _layer2_body(q_ref, dg_ref, h_ref, wl_ref, wr_ref, b2_ref, o_ref):
  d = dg_ref[0] + dg_ref[1]
  inv = 1.0 / jnp.maximum(d, 1.0)
  agg = (q_ref[0] + q_ref[1]) * inv
  o_ref[...] = (jnp.dot(agg, wl_ref[...], preferred_element_type=jnp.float32)
                + jnp.dot(h_ref[...], wr_ref[...],
                          preferred_element_type=jnp.float32)
                + b2_ref[...])


def _layer2(q, deg_col, h, W2_l, W2_r, b2):
  blk = 512
  grid = NP // blk
  return pl.pallas_call(
      _layer2_body,
      grid=(grid,),
      in_specs=[
          pl.BlockSpec((2, blk, D_HID), lambda i: (0, i, 0)),
          pl.BlockSpec((2, blk, 1), lambda i: (0, i, 0)),
          pl.BlockSpec((blk, D_HID), lambda i: (i, 0)),
          pl.BlockSpec((D_HID, D_OUT), lambda i: (0, 0)),
          pl.BlockSpec((D_HID, D_OUT), lambda i: (0, 0)),
          pl.BlockSpec((1, D_OUT), lambda i: (0, 0)),
      ],
      out_specs=pl.BlockSpec((blk, D_OUT), lambda i: (i, 0)),
      out_shape=jax.ShapeDtypeStruct((NP, D_OUT), jnp.float32),
  )(q, deg_col, h, W2_l, W2_r, b2.reshape(1, D_OUT))


def kernel(x, edge_index, W1_l, b1, W1_r, W2_l, b2, W2_r):
  src = edge_index[0].astype(jnp.int32)
  dst = edge_index[1].astype(jnp.int32)
  pad = EP - E
  srcp = jnp.concatenate([src, jnp.zeros((pad,), jnp.int32)]).reshape(NW, TK, B)
  dstp = jnp.concatenate([dst, jnp.full((pad,), DUMMY, jnp.int32)]
                         ).reshape(NW, TK, B)

  y1, z1 = _layer1_proj(x, W1_l, W1_r, b1)
  p1, deg = _seg_sum_deg(y1, srcp, dstp)
  deg_col = deg.reshape(2, NP, 1)
  h = _combine1(p1, deg_col, z1)
  p2 = jax.tree.leaves(_seg_sum(h, srcp, dstp))[0]
  out = _layer2(p2, deg_col, h, W2_l, W2_r, b2)
  return out[:N]


# speedup vs baseline: 3.8234x; 1.1097x over previous
import functools

import jax
import jax.numpy as jnp
from jax import lax
from jax.experimental import pallas as pl
from jax.experimental.pallas import tpu as pltpu
from jax.experimental.pallas import tpu_sc as plsc

N = 10000
E = 160000
D_IN = 256
D_HID = 128
D_OUT = 256

NP = 10240
DUMMY = N
NW = 32
TK = 40
B = 128
EP = NW * TK * B
RPT = NP // 16


def _make_seg_sum(with_deg: bool):
  mesh = plsc.VectorSubcoreMesh(core_axis_name="c", subcore_axis_name="s")
  out_type = [jax.ShapeDtypeStruct((2, NP, D_HID), jnp.float32)]
  if with_deg:
    out_type.append(jax.ShapeDtypeStruct((2, NP), jnp.float32))
  scratch = [
      pltpu.VMEM((TK, B), jnp.int32),
      pltpu.VMEM((TK, B), jnp.int32),
      pltpu.VMEM((B, D_HID), jnp.float32),
      pltpu.VMEM((B, D_HID), jnp.float32),
      pltpu.VMEM((B,), jnp.float32),
      pltpu.VMEM((B,), jnp.float32),
      pltpu.VMEM_SHARED((NP, D_HID), jnp.float32),
      pltpu.VMEM_SHARED((NP,), jnp.float32),
      pltpu.SemaphoreType.DMA,
      pltpu.SemaphoreType.DMA,
  ]

  def body(y_hbm, src_hbm, dst_hbm, *rest):
    if with_deg:
      (out_rows, out_deg, src_v, dst_v, rows_v0, rows_v1, ones_v, zeros_v,
       acc_sh, deg_sh, sem0, sem1) = rest
    else:
      (out_rows, src_v, dst_v, rows_v0, rows_v1, ones_v, zeros_v,
       acc_sh, deg_sh, sem0, sem1) = rest
      out_deg = None
    rows_v = rows_v0
    cid = lax.axis_index("c")
    sid = lax.axis_index("s")
    w = sid * 2 + cid

    one16 = jnp.ones((16,), jnp.float32)
    zero16 = jnp.zeros((16,), jnp.float32)
    for jj in range(B // 16):
      ones_v[pl.ds(jj * 16, 16)] = one16
      zeros_v[pl.ds(jj * 16, 16)] = zero16

    def zero_rows(i, carry):
      for jj in range(D_HID // 16):
        rows_v[i, pl.ds(jj * 16, 16)] = zero16
      return carry
    lax.fori_loop(0, B, zero_rows, 0)

    base = sid * RPT
    for bb in range(RPT // B):
      pltpu.sync_copy(rows_v, acc_sh.at[pl.ds(base + bb * B, B)])
      pltpu.sync_copy(zeros_v, deg_sh.at[pl.ds(base + bb * B, B)])

    pltpu.sync_copy(src_hbm.at[w], src_v)
    pltpu.sync_copy(dst_hbm.at[w], dst_v)
    plsc.subcore_barrier()

    g0 = pltpu.async_copy(y_hbm.at[src_v.at[0]], rows_v0, sem0)
    g1 = pltpu.async_copy(y_hbm.at[src_v.at[1]], rows_v1, sem1)

    def halfstep(j, buf, sem):
      pltpu.make_async_copy(y_hbm.at[src_v.at[j]], buf, sem).wait()
      pltpu.sync_copy(buf, acc_sh.at[dst_v.at[j]], add=True)
      if with_deg:
        pltpu.sync_copy(ones_v, deg_sh.at[dst_v.at[j]], add=True)

      @pl.when(j + 2 < TK)
      def _():
        pltpu.async_copy(y_hbm.at[src_v.at[j + 2]], buf, sem)

    def step(t, carry):
      halfstep(2 * t, rows_v0, sem0)
      halfstep(2 * t + 1, rows_v1, sem1)
      return carry
    lax.fori_loop(0, TK // 2, step, 0)

    plsc.subcore_barrier()
    pltpu.sync_copy(acc_sh.at[pl.ds(base, RPT)],
                    out_rows.at[cid, pl.ds(base, RPT)])
    if with_deg:
      pltpu.sync_copy(deg_sh.at[pl.ds(base, RPT)],
                      out_deg.at[cid, pl.ds(base, RPT)])

  return pl.kernel(body, mesh=mesh, out_type=out_type, scratch_types=scratch)


_seg_sum_deg = _make_seg_sum(with_deg=True)
_seg_sum = _make_seg_sum(with_deg=False)


def _proj_body(x_ref, wl_ref, wr_ref, b1_ref, y_ref, z_ref):
  x = x_ref[...]
  y_ref[...] = jnp.dot(x, wl_ref[...], preferred_element_type=jnp.float32)
  z_ref[...] = (jnp.dot(x, wr_ref[...], preferred_element_type=jnp.float32)
                + b1_ref[...])


def _layer1_proj(x, W1_l, W1_r, b1):
  blk = 400
  grid = N // blk
  return pl.pallas_call(
      _proj_body,
      grid=(grid,),
      in_specs=[
          pl.BlockSpec((blk, D_IN), lambda i: (i, 0)),
          pl.BlockSpec((D_IN, D_HID), lambda i: (0, 0)),
          pl.BlockSpec((D_IN, D_HID), lambda i: (0, 0)),
          pl.BlockSpec((1, D_HID), lambda i: (0, 0)),
      ],
      out_specs=[
          pl.BlockSpec((blk, D_HID), lambda i: (i, 0)),
          pl.BlockSpec((blk, D_HID), lambda i: (i, 0)),
      ],
      out_shape=[
          jax.ShapeDtypeStruct((NP, D_HID), jnp.float32),
          jax.ShapeDtypeStruct((NP, D_HID), jnp.float32),
      ],
  )(x, W1_l, W1_r, b1.reshape(1, D_HID))


def _combine1_body(p_ref, dg_ref, z_ref, h_ref):
  s = p_ref[0] + p_ref[1]
  d = dg_ref[0] + dg_ref[1]
  inv = 1.0 / jnp.maximum(d, 1.0)
  h_ref[...] = jnp.maximum(s * inv + z_ref[...], 0.0)


def _combine1(p, deg_col, z):
  blk = 512
  grid = NP // blk
  return pl.pallas_call(
      _combine1_body,
      grid=(grid,),
      in_specs=[
          pl.BlockSpec((2, blk, D_HID), lambda i: (0, i, 0)),
          pl.BlockSpec((2, blk, 1), lambda i: (0, i, 0)),
          pl.BlockSpec((blk, D_HID), lambda i: (i, 0)),
      ],
      out_specs=pl.BlockSpec((blk, D_HID), lambda i: (i, 0)),
      out_shape=jax.ShapeDtypeStruct((NP, D_HID), jnp.float32),
  )(p, deg_col, z)


def _layer2_body(q_ref, dg_ref, h_ref, wl_ref, wr_ref, b2_ref, o_ref):
  d = dg_ref[0] + dg_ref[1]
  inv = 1.0 / jnp.maximum(d, 1.0)
  agg = (q_ref[0] + q_ref[1]) * inv
  o_ref[...] = (jnp.dot(agg, wl_ref[...], preferred_element_type=jnp.float32)
                + jnp.dot(h_ref[...], wr_ref[...],
                          preferred_element_type=jnp.float32)
                + b2_ref[...])


def _layer2(q, deg_col, h, W2_l, W2_r, b2):
  blk = 512
  grid = NP // blk
  return pl.pallas_call(
      _layer2_body,
      grid=(grid,),
      in_specs=[
          pl.BlockSpec((2, blk, D_HID), lambda i: (0, i, 0)),
          pl.BlockSpec((2, blk, 1), lambda i: (0, i, 0)),
          pl.BlockSpec((blk, D_HID), lambda i: (i, 0)),
          pl.BlockSpec((D_HID, D_OUT), lambda i: (0, 0)),
          pl.BlockSpec((D_HID, D_OUT), lambda i: (0, 0)),
          pl.BlockSpec((1, D_OUT), lambda i: (0, 0)),
      ],
      out_specs=pl.BlockSpec((blk, D_OUT), lambda i: (i, 0)),
      out_shape=jax.ShapeDtypeStruct((NP, D_OUT), jnp.float32),
  )(q, deg_col, h, W2_l, W2_r, b2.reshape(1, D_OUT))


def kernel(x, edge_index, W1_l, b1, W1_r, W2_l, b2, W2_r):
  src = edge_index[0].astype(jnp.int32)
  dst = edge_index[1].astype(jnp.int32)
  pad = EP - E
  srcp = jnp.concatenate([src, jnp.zeros((pad,), jnp.int32)]).reshape(NW, TK, B)
  dstp = jnp.concatenate([dst, jnp.full((pad,), DUMMY, jnp.int32)]
                         ).reshape(NW, TK, B)

  y1, z1 = _layer1_proj(x, W1_l, W1_r, b1)
  p1, deg = _seg_sum_deg(y1, srcp, dstp)
  deg_col = deg.reshape(2, NP, 1)
  h = _combine1(p1, deg_col, z1)
  p2 = jax.tree.leaves(_seg_sum(h, srcp, dstp))[0]
  out = _layer2(p2, deg_col, h, W2_l, W2_r, b2)
  return out[:N]


# speedup vs baseline: 3.9374x; 1.0298x over previous
import functools

import jax
import jax.numpy as jnp
from jax import lax
from jax.experimental import pallas as pl
from jax.experimental.pallas import tpu as pltpu
from jax.experimental.pallas import tpu_sc as plsc

N = 10000
E = 160000
D_IN = 256
D_HID = 128
D_OUT = 256

NP = 10240
NW = 32
TK = 40
B = 128
EP = NW * TK * B
RPT = NP // 16


def _make_seg_sum(with_deg: bool):
  mesh = plsc.VectorSubcoreMesh(core_axis_name="c", subcore_axis_name="s")
  out_type = [jax.ShapeDtypeStruct((2, NP, D_HID), jnp.float32)]
  if with_deg:
    out_type.append(jax.ShapeDtypeStruct((2, NP), jnp.float32))
  scratch = [
      pltpu.VMEM((TK, B), jnp.int32),
      pltpu.VMEM((TK, B), jnp.int32),
      pltpu.VMEM((B, D_HID), jnp.float32),
      pltpu.VMEM((B, D_HID), jnp.float32),
      pltpu.VMEM((B,), jnp.float32),
      pltpu.VMEM((B,), jnp.float32),
      pltpu.VMEM_SHARED((NP, D_HID), jnp.float32),
      pltpu.VMEM_SHARED((NP,), jnp.float32),
      pltpu.SemaphoreType.DMA,
      pltpu.SemaphoreType.DMA,
  ]

  def body(y_hbm, src_hbm, dst_hbm, *rest):
    if with_deg:
      (out_rows, out_deg, src_v, dst_v, rows_v0, rows_v1, ones_v, zeros_v,
       acc_sh, deg_sh, sem0, sem1) = rest
    else:
      (out_rows, src_v, dst_v, rows_v0, rows_v1, ones_v, zeros_v,
       acc_sh, deg_sh, sem0, sem1) = rest
      out_deg = None
    rows_v = rows_v0
    cid = lax.axis_index("c")
    sid = lax.axis_index("s")
    w = sid * 2 + cid

    one16 = jnp.ones((16,), jnp.float32)
    zero16 = jnp.zeros((16,), jnp.float32)
    for jj in range(B // 16):
      ones_v[pl.ds(jj * 16, 16)] = one16
      zeros_v[pl.ds(jj * 16, 16)] = zero16

    def zero_rows(i, carry):
      for jj in range(D_HID // 16):
        rows_v[i, pl.ds(jj * 16, 16)] = zero16
      return carry
    lax.fori_loop(0, B, zero_rows, 0)

    base = sid * RPT
    for bb in range(RPT // B):
      pltpu.sync_copy(rows_v, acc_sh.at[pl.ds(base + bb * B, B)])
      pltpu.sync_copy(zeros_v, deg_sh.at[pl.ds(base + bb * B, B)])

    pltpu.sync_copy(src_hbm.at[w], src_v)
    pltpu.sync_copy(dst_hbm.at[w], dst_v)
    plsc.subcore_barrier()

    g0 = pltpu.async_copy(y_hbm.at[src_v.at[0]], rows_v0, sem0)
    g1 = pltpu.async_copy(y_hbm.at[src_v.at[1]], rows_v1, sem1)

    def halfstep(j, buf, sem):
      pltpu.make_async_copy(y_hbm.at[src_v.at[j]], buf, sem).wait()
      pltpu.sync_copy(buf, acc_sh.at[dst_v.at[j]], add=True)
      if with_deg:
        pltpu.sync_copy(ones_v, deg_sh.at[dst_v.at[j]], add=True)

      @pl.when(j + 2 < TK)
      def _():
        pltpu.async_copy(y_hbm.at[src_v.at[j + 2]], buf, sem)

    def step(t, carry):
      halfstep(2 * t, rows_v0, sem0)
      halfstep(2 * t + 1, rows_v1, sem1)
      return carry
    lax.fori_loop(0, TK // 2, step, 0)

    plsc.subcore_barrier()
    pltpu.sync_copy(acc_sh.at[pl.ds(base, RPT)],
                    out_rows.at[cid, pl.ds(base, RPT)])
    if with_deg:
      pltpu.sync_copy(deg_sh.at[pl.ds(base, RPT)],
                      out_deg.at[cid, pl.ds(base, RPT)])

  return pl.kernel(body, mesh=mesh, out_type=out_type, scratch_types=scratch)


_seg_sum_deg = _make_seg_sum(with_deg=True)
_seg_sum = _make_seg_sum(with_deg=False)


def _proj_body(x_ref, wl_ref, wr_ref, b1_ref, y_ref, z_ref):
  x = x_ref[...]
  y_ref[...] = jnp.dot(x, wl_ref[...], preferred_element_type=jnp.float32)
  z_ref[...] = (jnp.dot(x, wr_ref[...], preferred_element_type=jnp.float32)
                + b1_ref[...])


def _layer1_proj(x, W1_l, W1_r, b1):
  blk = 400
  grid = N // blk
  return pl.pallas_call(
      _proj_body,
      grid=(grid,),
      in_specs=[
          pl.BlockSpec((blk, D_IN), lambda i: (i, 0)),
          pl.BlockSpec((D_IN, D_HID), lambda i: (0, 0)),
          pl.BlockSpec((D_IN, D_HID), lambda i: (0, 0)),
          pl.BlockSpec((1, D_HID), lambda i: (0, 0)),
      ],
      out_specs=[
          pl.BlockSpec((blk, D_HID), lambda i: (i, 0)),
          pl.BlockSpec((blk, D_HID), lambda i: (i, 0)),
      ],
      out_shape=[
          jax.ShapeDtypeStruct((NP, D_HID), jnp.float32),
          jax.ShapeDtypeStruct((NP, D_HID), jnp.float32),
      ],
  )(x, W1_l, W1_r, b1.reshape(1, D_HID))


def _combine1_body(p_ref, dg_ref, z_ref, h_ref):
  s = p_ref[0] + p_ref[1]
  d = dg_ref[0] + dg_ref[1]
  inv = 1.0 / jnp.maximum(d, 1.0)
  h_ref[...] = jnp.maximum(s * inv + z_ref[...], 0.0)


def _combine1(p, deg_col, z):
  blk = 512
  grid = NP // blk
  return pl.pallas_call(
      _combine1_body,
      grid=(grid,),
      in_specs=[
          pl.BlockSpec((2, blk, D_HID), lambda i: (0, i, 0)),
          pl.BlockSpec((2, blk, 1), lambda i: (0, i, 0)),
          pl.BlockSpec((blk, D_HID), lambda i: (i, 0)),
      ],
      out_specs=pl.BlockSpec((blk, D_HID), lambda i: (i, 0)),
      out_shape=jax.ShapeDtypeStruct((NP, D_HID), jnp.float32),
  )(p, deg_col, z)


def _layer2_body(q_ref, dg_ref, h_ref, wl_ref, wr_ref, b2_ref, o_ref):
  d = dg_ref[0] + dg_ref[1]
  inv = 1.0 / jnp.maximum(d, 1.0)
  agg = (q_ref[0] + q_ref[1]) * inv
  o_ref[...] = (jnp.dot(agg, wl_ref[...], preferred_element_type=jnp.float32)
                + jnp.dot(h_ref[...], wr_ref[...],
                          preferred_element_type=jnp.float32)
                + b2_ref[...])


def _layer2(q, deg_col, h, W2_l, W2_r, b2):
  blk = 512
  grid = NP // blk
  return pl.pallas_call(
      _layer2_body,
      grid=(grid,),
      in_specs=[
          pl.BlockSpec((2, blk, D_HID), lambda i: (0, i, 0)),
          pl.BlockSpec((2, blk, 1), lambda i: (0, i, 0)),
          pl.BlockSpec((blk, D_HID), lambda i: (i, 0)),
          pl.BlockSpec((D_HID, D_OUT), lambda i: (0, 0)),
          pl.BlockSpec((D_HID, D_OUT), lambda i: (0, 0)),
          pl.BlockSpec((1, D_OUT), lambda i: (0, 0)),
      ],
      out_specs=pl.BlockSpec((blk, D_OUT), lambda i: (i, 0)),
      out_shape=jax.ShapeDtypeStruct((NP, D_OUT), jnp.float32),
  )(q, deg_col, h, W2_l, W2_r, b2.reshape(1, D_OUT))


def kernel(x, edge_index, W1_l, b1, W1_r, W2_l, b2, W2_r):
  src = edge_index[0].astype(jnp.int32)
  dst = edge_index[1].astype(jnp.int32)
  pad = EP - E
  srcp = jnp.concatenate([src, jnp.zeros((pad,), jnp.int32)]).reshape(NW, TK, B)
  pad_dst = N + (jnp.arange(pad, dtype=jnp.int32) % (NP - N))
  dstp = jnp.concatenate([dst, pad_dst]).reshape(NW, TK, B)

  y1, z1 = _layer1_proj(x, W1_l, W1_r, b1)
  p1, deg = _seg_sum_deg(y1, srcp, dstp)
  deg_col = deg.reshape(2, NP, 1)
  h = _combine1(p1, deg_col, z1)
  p2 = jax.tree.leaves(_seg_sum(h, srcp, dstp))[0]
  out = _layer2(p2, deg_col, h, W2_l, W2_r, b2)
  return out[:N]


# speedup vs baseline: 4.2003x; 1.0668x over previous
import functools

import jax
import jax.numpy as jnp
from jax import lax
from jax.experimental import pallas as pl
from jax.experimental.pallas import tpu as pltpu
from jax.experimental.pallas import tpu_sc as plsc

N = 10000
E = 160000
D_IN = 256
D_HID = 128
D_OUT = 256

NP = 10240
B = 128
TKA = 56
TKB = 24
EA = 16 * TKA * B
EB = 16 * TKB * B
RPT = NP // 16


def _make_seg_sum(with_deg: bool):
  mesh = plsc.VectorSubcoreMesh(core_axis_name="c", subcore_axis_name="s")
  out_type = [jax.ShapeDtypeStruct((2, NP, D_HID), jnp.float32)]
  if with_deg:
    out_type.append(jax.ShapeDtypeStruct((2, NP), jnp.float32))
  scratch = [
      pltpu.VMEM((TKA, B), jnp.int32),
      pltpu.VMEM((TKA, B), jnp.int32),
      pltpu.VMEM((B, D_HID), jnp.float32),
      pltpu.VMEM((B, D_HID), jnp.float32),
      pltpu.VMEM((B,), jnp.float32),
      pltpu.VMEM((B,), jnp.float32),
      pltpu.VMEM_SHARED((NP, D_HID), jnp.float32),
  ]
  if with_deg:
    scratch.append(pltpu.VMEM_SHARED((NP,), jnp.float32))
  scratch += [
      pltpu.SemaphoreType.DMA,
      pltpu.SemaphoreType.DMA,
  ]

  def body(y_hbm, srca_hbm, dsta_hbm, srcb_hbm, dstb_hbm, *rest):
    if with_deg:
      (out_rows, out_deg, src_v, dst_v, rows_v0, rows_v1, ones_v, zeros_v,
       acc_sh, deg_sh, sem0, sem1) = rest
    else:
      (out_rows, src_v, dst_v, rows_v0, rows_v1, ones_v, zeros_v,
       acc_sh, sem0, sem1) = rest
      out_deg = None
      deg_sh = None
    rows_v = rows_v0
    cid = lax.axis_index("c")
    sid = lax.axis_index("s")

    one16 = jnp.ones((16,), jnp.float32)
    zero16 = jnp.zeros((16,), jnp.float32)
    for jj in range(B // 16):
      ones_v[pl.ds(jj * 16, 16)] = one16
      zeros_v[pl.ds(jj * 16, 16)] = zero16

    def zero_rows(i, carry):
      for jj in range(D_HID // 16):
        rows_v[i, pl.ds(jj * 16, 16)] = zero16
      return carry
    lax.fori_loop(0, B, zero_rows, 0)

    base = sid * RPT
    for bb in range(RPT // B):
      pltpu.sync_copy(rows_v, acc_sh.at[pl.ds(base + bb * B, B)])
      if with_deg:
        pltpu.sync_copy(zeros_v, deg_sh.at[pl.ds(base + bb * B, B)])

    @pl.when(cid == 0)
    def _():
      pltpu.sync_copy(srca_hbm.at[sid], src_v)
      pltpu.sync_copy(dsta_hbm.at[sid], dst_v)

    @pl.when(cid == 1)
    def _():
      pltpu.sync_copy(srcb_hbm.at[sid], src_v.at[pl.ds(0, TKB)])
      pltpu.sync_copy(dstb_hbm.at[sid], dst_v.at[pl.ds(0, TKB)])

    plsc.subcore_barrier()

    def run_batches(tk):
      pltpu.async_copy(y_hbm.at[src_v.at[0]], rows_v0, sem0)
      pltpu.async_copy(y_hbm.at[src_v.at[1]], rows_v1, sem1)

      def halfstep(j, buf, sem):
        pltpu.make_async_copy(y_hbm.at[src_v.at[j]], buf, sem).wait()
        pltpu.sync_copy(buf, acc_sh.at[dst_v.at[j]], add=True)
        if with_deg:
          pltpu.sync_copy(ones_v, deg_sh.at[dst_v.at[j]], add=True)

        @pl.when(j + 2 < tk)
        def _():
          pltpu.async_copy(y_hbm.at[src_v.at[j + 2]], buf, sem)

      def step(t, carry):
        halfstep(2 * t, rows_v0, sem0)
        halfstep(2 * t + 1, rows_v1, sem1)
        return carry
      lax.fori_loop(0, tk // 2, step, 0)

    @pl.when(cid == 0)
    def _():
      run_batches(TKA)

    @pl.when(cid == 1)
    def _():
      run_batches(TKB)

    plsc.subcore_barrier()
    pltpu.sync_copy(acc_sh.at[pl.ds(base, RPT)],
                    out_rows.at[cid, pl.ds(base, RPT)])
    if with_deg:
      pltpu.sync_copy(deg_sh.at[pl.ds(base, RPT)],
                      out_deg.at[cid, pl.ds(base, RPT)])

  return pl.kernel(body, mesh=mesh, out_type=out_type, scratch_types=scratch)


_seg_sum_deg = _make_seg_sum(with_deg=True)
_seg_sum = _make_seg_sum(with_deg=False)


def _proj_body(x_ref, wl_ref, wr_ref, b1_ref, y_ref, z_ref):
  x = x_ref[...]
  y_ref[...] = jnp.dot(x, wl_ref[...], preferred_element_type=jnp.float32)
  z_ref[...] = (jnp.dot(x, wr_ref[...], preferred_element_type=jnp.float32)
                + b1_ref[...])


def _layer1_proj(x, W1_l, W1_r, b1):
  blk = 400
  grid = N // blk
  return pl.pallas_call(
      _proj_body,
      grid=(grid,),
      in_specs=[
          pl.BlockSpec((blk, D_IN), lambda i: (i, 0)),
          pl.BlockSpec((D_IN, D_HID), lambda i: (0, 0)),
          pl.BlockSpec((D_IN, D_HID), lambda i: (0, 0)),
          pl.BlockSpec((1, D_HID), lambda i: (0, 0)),
      ],
      out_specs=[
          pl.BlockSpec((blk, D_HID), lambda i: (i, 0)),
          pl.BlockSpec((blk, D_HID), lambda i: (i, 0)),
      ],
      out_shape=[
          jax.ShapeDtypeStruct((NP, D_HID), jnp.float32),
          jax.ShapeDtypeStruct((NP, D_HID), jnp.float32),
      ],
  )(x, W1_l, W1_r, b1.reshape(1, D_HID))


def _combine1_body(p_ref, dg_ref, z_ref, h_ref):
  s = p_ref[0] + p_ref[1]
  d = dg_ref[0] + dg_ref[1]
  inv = 1.0 / jnp.maximum(d, 1.0)
  h_ref[...] = jnp.maximum(s * inv + z_ref[...], 0.0)


def _combine1(p, deg_col, z):
  blk = 512
  grid = NP // blk
  return pl.pallas_call(
      _combine1_body,
      grid=(grid,),
      in_specs=[
          pl.BlockSpec((2, blk, D_HID), lambda i: (0, i, 0)),
          pl.BlockSpec((2, blk, 1), lambda i: (0, i, 0)),
          pl.BlockSpec((blk, D_HID), lambda i: (i, 0)),
      ],
      out_specs=pl.BlockSpec((blk, D_HID), lambda i: (i, 0)),
      out_shape=jax.ShapeDtypeStruct((NP, D_HID), jnp.float32),
  )(p, deg_col, z)


def _layer2_body(q_ref, dg_ref, h_ref, wl_ref, wr_ref, b2_ref, o_ref):
  d = dg_ref[0] + dg_ref[1]
  inv = 1.0 / jnp.maximum(d, 1.0)
  agg = (q_ref[0] + q_ref[1]) * inv
  o_ref[...] = (jnp.dot(agg, wl_ref[...], preferred_element_type=jnp.float32)
                + jnp.dot(h_ref[...], wr_ref[...],
                          preferred_element_type=jnp.float32)
                + b2_ref[...])


def _layer2(q, deg_col, h, W2_l, W2_r, b2):
  blk = 512
  grid = NP // blk
  return pl.pallas_call(
      _layer2_body,
      grid=(grid,),
      in_specs=[
          pl.BlockSpec((2, blk, D_HID), lambda i: (0, i, 0)),
          pl.BlockSpec((2, blk, 1), lambda i: (0, i, 0)),
          pl.BlockSpec((blk, D_HID), lambda i: (i, 0)),
          pl.BlockSpec((D_HID, D_OUT), lambda i: (0, 0)),
          pl.BlockSpec((D_HID, D_OUT), lambda i: (0, 0)),
          pl.BlockSpec((1, D_OUT), lambda i: (0, 0)),
      ],
      out_specs=pl.BlockSpec((blk, D_OUT), lambda i: (i, 0)),
      out_shape=jax.ShapeDtypeStruct((NP, D_OUT), jnp.float32),
  )(q, deg_col, h, W2_l, W2_r, b2.reshape(1, D_OUT))


def kernel(x, edge_index, W1_l, b1, W1_r, W2_l, b2, W2_r):
  src = edge_index[0].astype(jnp.int32)
  dst = edge_index[1].astype(jnp.int32)
  pad = EA + EB - E
  srca = src[:EA].reshape(16, TKA, B)
  dsta = dst[:EA].reshape(16, TKA, B)
  srcb = jnp.concatenate([src[EA:], jnp.zeros((pad,), jnp.int32)]
                         ).reshape(16, TKB, B)
  pad_dst = N + (jnp.arange(pad, dtype=jnp.int32) % (NP - N))
  dstb = jnp.concatenate([dst[EA:], pad_dst]).reshape(16, TKB, B)

  y1, z1 = _layer1_proj(x, W1_l, W1_r, b1)
  p1, deg = _seg_sum_deg(y1, srca, dsta, srcb, dstb)
  deg_col = deg.reshape(2, NP, 1)
  h = _combine1(p1, deg_col, z1)
  p2 = jax.tree.leaves(_seg_sum(h, srca, dsta, srcb, dstb))[0]
  out = _layer2(p2, deg_col, h, W2_l, W2_r, b2)
  return out[:N]
